# baseline (speedup 1.0000x reference)
import math

import jax
import jax.numpy as jnp
from jax.experimental import pallas as pl

V, D, DF = 8192, 256, 1024
NB, R, NN, KTOP, H, L = 32, 32, 64, 8, 4, 2
B, S = 2, 2048
DH = D // H
T = B * S

_NEG = -1e30


def _ln(x, s, b):
    m = jnp.mean(x, axis=-1, keepdims=True)
    xc = x - m
    v = jnp.mean(xc * xc, axis=-1, keepdims=True)
    return xc * jax.lax.rsqrt(v + 1e-5) * s + b


_TBQ = 512


def _qkv_body(x_ref, s_ref, b_ref, qw_ref, qb_ref, kw_ref, kb_ref,
              vw_ref, vb_ref, n_ref, q_ref, k_ref, v_ref):
    x = x_ref[...]
    n = _ln(x, s_ref[...], b_ref[...])
    n_ref[...] = n
    q_ref[...] = jnp.dot(n, qw_ref[...], preferred_element_type=jnp.float32) + qb_ref[...]
    k_ref[...] = jnp.dot(n, kw_ref[...], preferred_element_type=jnp.float32) + kb_ref[...]
    v_ref[...] = jnp.dot(n, vw_ref[...], preferred_element_type=jnp.float32) + vb_ref[...]


def _qkv(x, s, b, qw, qb, kw, kb, vw, vb):
    tok = pl.BlockSpec((_TBQ, D), lambda i: (i, 0))
    full = pl.BlockSpec((D, D), lambda i: (0, 0))
    vec = pl.BlockSpec((1, D), lambda i: (0, 0))
    return pl.pallas_call(
        _qkv_body,
        grid=(T // _TBQ,),
        in_specs=[tok, vec, vec, full, vec, full, vec, full, vec],
        out_specs=[tok, tok, tok, tok],
        out_shape=[jax.ShapeDtypeStruct((T, D), jnp.float32)] * 4,
    )(x, s, b, qw, qb, kw, kb, vw, vb)


_QB = 512


def _attn_body(q_ref, k_ref, v_ref, o_ref):
    qi = pl.program_id(1)
    q = q_ref[0]
    k = k_ref[0]
    v = v_ref[0]
    s = jax.lax.dot_general(q, k, (((1,), (1,)), ((), ())),
                            preferred_element_type=jnp.float32)
    s = s * (1.0 / math.sqrt(DH))
    row = qi * _QB + jax.lax.broadcasted_iota(jnp.int32, (_QB, S), 0)
    col = jax.lax.broadcasted_iota(jnp.int32, (_QB, S), 1)
    keep = col <= row
    s = jnp.where(keep, s, _NEG)
    mx = jnp.max(s, axis=-1, keepdims=True)
    e = jnp.exp(s - mx)
    e = jnp.where(keep, e, 0.0)
    p = e / jnp.sum(e, axis=-1, keepdims=True)
    o_ref[0] = jnp.dot(p, v, preferred_element_type=jnp.float32)


def _attn(q, k, v):
    qspec = pl.BlockSpec((1, _QB, DH), lambda bh, qi: (bh, qi, 0))
    kspec = pl.BlockSpec((1, S, DH), lambda bh, qi: (bh, 0, 0))
    return pl.pallas_call(
        _attn_body,
        grid=(B * H, S // _QB),
        in_specs=[qspec, kspec, kspec],
        out_specs=qspec,
        out_shape=jax.ShapeDtypeStruct((B * H, S, DH), jnp.float32),
    )(q, k, v)


_TBR = 512


def _router_body(n_ref, c_ref, sw1_ref, sw2_ref, sb_ref, rec_ref, be_ref,
                 wr_ref):
    n = n_ref[...]
    c = c_ref[...]
    query = (jnp.dot(n, sw1_ref[...], preferred_element_type=jnp.float32)
             + jnp.dot(c, sw2_ref[...], preferred_element_type=jnp.float32)
             + sb_ref[...])
    rec = rec_ref[...]
    rec_sm = jax.nn.softmax(rec, axis=-1)
    nemb = jnp.dot(rec_sm, be_ref[...], preferred_element_type=jnp.float32)
    scores = jax.lax.dot_general(query, nemb, (((1,), (1,)), ((), ())),
                                 preferred_element_type=jnp.float32)
    idx = jax.lax.broadcasted_iota(jnp.int32, scores.shape, 1)
    sel = jnp.zeros(scores.shape, jnp.bool_)
    work = scores
    for _ in range(KTOP):
        mx = jnp.max(work, axis=-1, keepdims=True)
        is_max = work == mx
        cand_idx = jnp.where(is_max, idx, NN)
        amin = jnp.min(cand_idx, axis=-1, keepdims=True)
        first = idx == amin
        sel = jnp.logical_or(sel, first)
        work = jnp.where(first, _NEG, work)
    smax = jnp.max(jnp.where(sel, scores, _NEG), axis=-1, keepdims=True)
    e = jnp.where(sel, jnp.exp(scores - smax), 0.0)
    w = e / jnp.sum(e, axis=-1, keepdims=True)
    wr_ref[...] = jnp.dot(w, rec_sm, preferred_element_type=jnp.float32)


def _router(n, c, sw1, sw2, sb, rec, be):
    tok = pl.BlockSpec((_TBR, D), lambda i: (i, 0))
    full = pl.BlockSpec((D, D), lambda i: (0, 0))
    vec = pl.BlockSpec((1, D), lambda i: (0, 0))
    recs = pl.BlockSpec((NN, NB), lambda i: (0, 0))
    bes = pl.BlockSpec((NB, D), lambda i: (0, 0))
    wrs = pl.BlockSpec((_TBR, NB), lambda i: (i, 0))
    return pl.pallas_call(
        _router_body,
        grid=(T // _TBR,),
        in_specs=[tok, tok, full, full, vec, recs, bes],
        out_specs=wrs,
        out_shape=jax.ShapeDtypeStruct((T, NB), jnp.float32),
    )(n, c, sw1, sw2, sb, rec, be)


_TBF = 64


def _tdot(a, b):
    return jax.lax.dot_general(a, b, (((0,), (0,)), ((), ())),
                               preferred_element_type=jnp.float32)


def _ffn_body(x_ref, wr_ref, s_ref, b_ref, a1_ref, a2_ref, b1_ref, b2_ref,
              wd_ref, wdb_ref, o_ref):
    x = x_ref[...]
    wr = wr_ref[...]
    wrt = wr.T
    n2 = _ln(x, s_ref[...], b_ref[...])

    xfi = n2.reshape(_TBF, 16, 16).transpose(1, 0, 2)
    xfm = xfi.reshape(16, _TBF * 16)
    wtj = jnp.broadcast_to(wrt[:, :, None], (NB, _TBF, 16)).reshape(NB, _TBF * 16)
    zt = (wtj[:, None, :] * xfm[None, :, :]).reshape(NB * 16, _TBF * 16)
    t1t = jnp.dot(a1_ref[...], zt, preferred_element_type=jnp.float32)
    ca2 = jnp.dot(a2_ref[...], wrt, preferred_element_type=jnp.float32)
    ca2 = ca2.reshape(R * 8, 16, _TBF).transpose(0, 2, 1)
    ca2 = ca2.reshape(R * 8, _TBF * 16)
    prod = (t1t.reshape(R, 8, 1, _TBF * 16)
            * ca2.reshape(R, 1, 8, _TBF * 16)).sum(axis=0)
    h3 = prod.reshape(8, 8, _TBF, 16).sum(axis=3)

    hfm = h3.transpose(0, 2, 1).reshape(8, _TBF * 8)
    wtj2 = jnp.broadcast_to(wrt[:, :, None], (NB, _TBF, 8)).reshape(NB, _TBF * 8)
    zbt = (wtj2[:, None, :] * hfm[None, :, :]).reshape(NB * 8, _TBF * 8)
    ut = jnp.dot(b1_ref[...], zbt, preferred_element_type=jnp.float32)
    cb2 = jnp.dot(b2_ref[...], wrt, preferred_element_type=jnp.float32)
    cb2 = cb2.reshape(R * 32, 8, _TBF).transpose(0, 2, 1)
    cb2 = cb2.reshape(R * 32, _TBF * 8)
    acc = jnp.zeros((32, 32, _TBF * 8), jnp.float32)
    for rc in range(8):
        rs = slice(rc * 4, (rc + 1) * 4)
        acc = acc + (ut.reshape(R, 32, 1, _TBF * 8)[rs]
                     * cb2.reshape(R, 1, 32, _TBF * 8)[rs]).sum(axis=0)
    out3 = acc.reshape(32, 32, _TBF, 8).sum(axis=3)
    ov = out3.reshape(DF, _TBF)

    g = 0.5 * ov * (1.0 + jax.lax.erf(ov * (1.0 / math.sqrt(2.0))))
    y = _tdot(g, wd_ref[...])
    o_ref[...] = x + y + wdb_ref[...]


def _ffn(x, wr, s, b, a1m, a2m, b1m, b2m, wd, wdb):
    tok = pl.BlockSpec((_TBF, D), lambda i: (i, 0))
    wrs = pl.BlockSpec((_TBF, NB), lambda i: (i, 0))
    vec = pl.BlockSpec((1, D), lambda i: (0, 0))
    a1s = pl.BlockSpec((R * 8, NB * 16), lambda i: (0, 0))
    a2s = pl.BlockSpec((R * 16 * 8, NB), lambda i: (0, 0))
    b1s = pl.BlockSpec((R * 32, NB * 8), lambda i: (0, 0))
    b2s = pl.BlockSpec((R * 8 * 32, NB), lambda i: (0, 0))
    wds = pl.BlockSpec((DF, D), lambda i: (0, 0))
    return pl.pallas_call(
        _ffn_body,
        grid=(T // _TBF,),
        in_specs=[tok, wrs, vec, vec, a1s, a2s, b1s, b2s, wds, vec],
        out_specs=tok,
        out_shape=jax.ShapeDtypeStruct((T, D), jnp.float32),
    )(x, wr, s, b, a1m, a2m, b1m, b2m, wd, wdb)


_TBH = 256


def _head_body(x_ref, s_ref, b_ref, te_ref, o_ref):
    xn = _ln(x_ref[...], s_ref[...], b_ref[...])
    o_ref[...] = jax.lax.dot_general(xn, te_ref[...], (((1,), (1,)), ((), ())),
                                     preferred_element_type=jnp.float32)


def _head(x, s, b, te):
    tok = pl.BlockSpec((_TBH, D), lambda i: (i, 0))
    vec = pl.BlockSpec((1, D), lambda i: (0, 0))
    tes = pl.BlockSpec((V, D), lambda i: (0, 0))
    outs = pl.BlockSpec((_TBH, V), lambda i: (i, 0))
    return pl.pallas_call(
        _head_body,
        grid=(T // _TBH,),
        in_specs=[tok, vec, vec, tes],
        out_specs=outs,
        out_shape=jax.ShapeDtypeStruct((T, V), jnp.float32),
    )(x, s, b, te)


def kernel(input_ids, params):
    te = params['token_emb']
    x = te[input_ids.reshape(-1)] + jnp.broadcast_to(
        params['pos_emb'][None], (B, S, D)).reshape(T, D)

    a1m = params['A1'].transpose(2, 3, 0, 1).reshape(R * 8, NB * 16)
    a2m = params['A2'].transpose(1, 3, 2, 0).reshape(R * 16 * 8, NB)
    b1m = params['B1'].transpose(2, 3, 0, 1).reshape(R * 32, NB * 8)
    b2m = params['B2'].transpose(1, 3, 2, 0).reshape(R * 8 * 32, NB)
    be = params['basis_emb']

    def v2(a):
        return a.reshape(1, -1)

    for p in params['layers']:
        n, q, k, v = _qkv(x, v2(p['n1s']), v2(p['n1b']), p['qw'], v2(p['qb']),
                          p['kw'], v2(p['kb']), p['vw'], v2(p['vb']))
        def heads(a):
            return a.reshape(B, S, H, DH).transpose(0, 2, 1, 3).reshape(B * H, S, DH)
        ctx = _attn(heads(q), heads(k), heads(v))
        ctx = ctx.reshape(B, H, S, DH).transpose(0, 2, 1, 3).reshape(T, D)
        wr = _router(n, ctx, p['sw'][:D], p['sw'][D:], v2(p['sb']),
                     p['recipes'], be)
        x = _ffn(x, wr, v2(p['n2s']), v2(p['n2b']), a1m, a2m, b1m, b2m,
                 p['wd'], v2(p['wdb']))

    logits = _head(x, v2(params['final_s']), v2(params['final_b']), te)
    return logits.reshape(B, S, V)


# speedup vs baseline: 3.9282x; 3.9282x over previous
import math

import jax
import jax.numpy as jnp
from jax.experimental import pallas as pl

V, D, DF = 8192, 256, 1024
NB, R, NN, KTOP, H, L = 32, 32, 64, 8, 4, 2
B, S = 2, 2048
DH = D // H
T = B * S

_NEG = -1e30


def _ln(x, s, b):
    m = jnp.mean(x, axis=-1, keepdims=True)
    xc = x - m
    v = jnp.mean(xc * xc, axis=-1, keepdims=True)
    return xc * jax.lax.rsqrt(v + 1e-5) * s + b


_TBQ = 512


def _qkv_body(x_ref, s_ref, b_ref, qw_ref, qb_ref, kw_ref, kb_ref,
              vw_ref, vb_ref, n_ref, q_ref, k_ref, v_ref):
    x = x_ref[...]
    n = _ln(x, s_ref[...], b_ref[...])
    n_ref[...] = n
    q_ref[...] = jnp.dot(n, qw_ref[...], preferred_element_type=jnp.float32) + qb_ref[...]
    k_ref[...] = jnp.dot(n, kw_ref[...], preferred_element_type=jnp.float32) + kb_ref[...]
    v_ref[...] = jnp.dot(n, vw_ref[...], preferred_element_type=jnp.float32) + vb_ref[...]


def _qkv(x, s, b, qw, qb, kw, kb, vw, vb):
    tok = pl.BlockSpec((_TBQ, D), lambda i: (i, 0))
    full = pl.BlockSpec((D, D), lambda i: (0, 0))
    vec = pl.BlockSpec((1, D), lambda i: (0, 0))
    return pl.pallas_call(
        _qkv_body,
        grid=(T // _TBQ,),
        in_specs=[tok, vec, vec, full, vec, full, vec, full, vec],
        out_specs=[tok, tok, tok, tok],
        out_shape=[jax.ShapeDtypeStruct((T, D), jnp.float32)] * 4,
    )(x, s, b, qw, qb, kw, kb, vw, vb)


_QB = 512


def _attn_body(q_ref, k_ref, v_ref, o_ref):
    qi = pl.program_id(1)
    q = q_ref[0]
    k = k_ref[0]
    v = v_ref[0]
    s = jax.lax.dot_general(q, k, (((1,), (1,)), ((), ())),
                            preferred_element_type=jnp.float32)
    s = s * (1.0 / math.sqrt(DH))
    row = qi * _QB + jax.lax.broadcasted_iota(jnp.int32, (_QB, S), 0)
    col = jax.lax.broadcasted_iota(jnp.int32, (_QB, S), 1)
    keep = col <= row
    s = jnp.where(keep, s, _NEG)
    mx = jnp.max(s, axis=-1, keepdims=True)
    e = jnp.exp(s - mx)
    e = jnp.where(keep, e, 0.0)
    p = e / jnp.sum(e, axis=-1, keepdims=True)
    o_ref[0] = jnp.dot(p, v, preferred_element_type=jnp.float32)


def _attn(q, k, v):
    qspec = pl.BlockSpec((1, _QB, DH), lambda bh, qi: (bh, qi, 0))
    kspec = pl.BlockSpec((1, S, DH), lambda bh, qi: (bh, 0, 0))
    return pl.pallas_call(
        _attn_body,
        grid=(B * H, S // _QB),
        in_specs=[qspec, kspec, kspec],
        out_specs=qspec,
        out_shape=jax.ShapeDtypeStruct((B * H, S, DH), jnp.float32),
    )(q, k, v)


_TBR = 512


def _router_body(n_ref, c_ref, sw1_ref, sw2_ref, sb_ref, rec_ref, be_ref,
                 wr_ref):
    n = n_ref[...]
    c = c_ref[...]
    query = (jnp.dot(n, sw1_ref[...], preferred_element_type=jnp.float32)
             + jnp.dot(c, sw2_ref[...], preferred_element_type=jnp.float32)
             + sb_ref[...])
    rec = rec_ref[...]
    rec_sm = jax.nn.softmax(rec, axis=-1)
    nemb = jnp.dot(rec_sm, be_ref[...], preferred_element_type=jnp.float32)
    scores = jax.lax.dot_general(query, nemb, (((1,), (1,)), ((), ())),
                                 preferred_element_type=jnp.float32)
    idx = jax.lax.broadcasted_iota(jnp.int32, scores.shape, 1)
    sel = jnp.zeros(scores.shape, jnp.bool_)
    work = scores
    for _ in range(KTOP):
        mx = jnp.max(work, axis=-1, keepdims=True)
        is_max = work == mx
        cand_idx = jnp.where(is_max, idx, NN)
        amin = jnp.min(cand_idx, axis=-1, keepdims=True)
        first = idx == amin
        sel = jnp.logical_or(sel, first)
        work = jnp.where(first, _NEG, work)
    smax = jnp.max(jnp.where(sel, scores, _NEG), axis=-1, keepdims=True)
    e = jnp.where(sel, jnp.exp(scores - smax), 0.0)
    w = e / jnp.sum(e, axis=-1, keepdims=True)
    wr_ref[...] = jnp.dot(w, rec_sm, preferred_element_type=jnp.float32)


def _router(n, c, sw1, sw2, sb, rec, be):
    tok = pl.BlockSpec((_TBR, D), lambda i: (i, 0))
    full = pl.BlockSpec((D, D), lambda i: (0, 0))
    vec = pl.BlockSpec((1, D), lambda i: (0, 0))
    recs = pl.BlockSpec((NN, NB), lambda i: (0, 0))
    bes = pl.BlockSpec((NB, D), lambda i: (0, 0))
    wrs = pl.BlockSpec((_TBR, NB), lambda i: (i, 0))
    return pl.pallas_call(
        _router_body,
        grid=(T // _TBR,),
        in_specs=[tok, tok, full, full, vec, recs, bes],
        out_specs=wrs,
        out_shape=jax.ShapeDtypeStruct((T, NB), jnp.float32),
    )(n, c, sw1, sw2, sb, rec, be)


_TBF = 128


def _tdot(a, b):
    return jax.lax.dot_general(a, b, (((0,), (0,)), ((), ())),
                               preferred_element_type=jnp.float32)


def _ffn_body(x_ref, wr_ref, s_ref, b_ref, a1_ref, a2_ref, b1_ref, b2_ref,
              wd_ref, wdb_ref, o_ref):
    x = x_ref[...]
    wr = wr_ref[...]
    wrt = wr.T
    n2 = _ln(x, s_ref[...], b_ref[...])

    xfm = n2.reshape(_TBF, 16, 16).transpose(1, 2, 0).reshape(16, 16 * _TBF)
    wtj = jnp.broadcast_to(wrt[:, None, :], (NB, 16, _TBF)).reshape(NB, 16 * _TBF)
    zt = (wtj[:, None, :] * xfm[None, :, :]).reshape(NB * 16, 16 * _TBF)
    t1t = jnp.dot(a1_ref[...], zt, preferred_element_type=jnp.float32)
    ca2 = jnp.dot(a2_ref[...], wrt, preferred_element_type=jnp.float32)
    ca2 = ca2.reshape(R * 8, 16 * _TBF)
    t4 = t1t.reshape(R, 8, 1, 16 * _TBF)
    c4 = ca2.reshape(R, 1, 8, 16 * _TBF)
    prod = jnp.zeros((8, 8, 16 * _TBF), jnp.float32)
    for rc in range(4):
        rs = slice(rc * 8, (rc + 1) * 8)
        prod = prod + (t4[rs] * c4[rs]).sum(axis=0)
    h3 = prod.reshape(8, 8, 16, _TBF).sum(axis=2)

    hfm = h3.reshape(8, 8 * _TBF)
    wtj2 = jnp.broadcast_to(wrt[:, None, :], (NB, 8, _TBF)).reshape(NB, 8 * _TBF)
    zbt = (wtj2[:, None, :] * hfm[None, :, :]).reshape(NB * 8, 8 * _TBF)
    ut = jnp.dot(b1_ref[...], zbt, preferred_element_type=jnp.float32)
    cb2 = jnp.dot(b2_ref[...], wrt, preferred_element_type=jnp.float32)
    cb2 = cb2.reshape(R * 32, 8 * _TBF)
    u4 = ut.reshape(R, 32, 1, 8 * _TBF)
    cb4 = cb2.reshape(R, 1, 32, 8 * _TBF)
    acc = jnp.zeros((32, 32, 8 * _TBF), jnp.float32)
    for rc in range(16):
        rs = slice(rc * 2, (rc + 1) * 2)
        acc = acc + (u4[rs] * cb4[rs]).sum(axis=0)
    out3 = acc.reshape(32, 32, 8, _TBF).sum(axis=2)
    ov = out3.reshape(DF, _TBF)

    g = 0.5 * ov * (1.0 + jax.lax.erf(ov * (1.0 / math.sqrt(2.0))))
    y = _tdot(g, wd_ref[...])
    o_ref[...] = x + y + wdb_ref[...]


def _ffn(x, wr, s, b, a1m, a2m, b1m, b2m, wd, wdb):
    tok = pl.BlockSpec((_TBF, D), lambda i: (i, 0))
    wrs = pl.BlockSpec((_TBF, NB), lambda i: (i, 0))
    vec = pl.BlockSpec((1, D), lambda i: (0, 0))
    a1s = pl.BlockSpec((R * 8, NB * 16), lambda i: (0, 0))
    a2s = pl.BlockSpec((R * 16 * 8, NB), lambda i: (0, 0))
    b1s = pl.BlockSpec((R * 32, NB * 8), lambda i: (0, 0))
    b2s = pl.BlockSpec((R * 8 * 32, NB), lambda i: (0, 0))
    wds = pl.BlockSpec((DF, D), lambda i: (0, 0))
    return pl.pallas_call(
        _ffn_body,
        grid=(T // _TBF,),
        in_specs=[tok, wrs, vec, vec, a1s, a2s, b1s, b2s, wds, vec],
        out_specs=tok,
        out_shape=jax.ShapeDtypeStruct((T, D), jnp.float32),
    )(x, wr, s, b, a1m, a2m, b1m, b2m, wd, wdb)


_TBH = 256


def _head_body(x_ref, s_ref, b_ref, te_ref, o_ref):
    xn = _ln(x_ref[...], s_ref[...], b_ref[...])
    o_ref[...] = jax.lax.dot_general(xn, te_ref[...], (((1,), (1,)), ((), ())),
                                     preferred_element_type=jnp.float32)


def _head(x, s, b, te):
    tok = pl.BlockSpec((_TBH, D), lambda i: (i, 0))
    vec = pl.BlockSpec((1, D), lambda i: (0, 0))
    tes = pl.BlockSpec((V, D), lambda i: (0, 0))
    outs = pl.BlockSpec((_TBH, V), lambda i: (i, 0))
    return pl.pallas_call(
        _head_body,
        grid=(T // _TBH,),
        in_specs=[tok, vec, vec, tes],
        out_specs=outs,
        out_shape=jax.ShapeDtypeStruct((T, V), jnp.float32),
    )(x, s, b, te)


def kernel(input_ids, params):
    te = params['token_emb']
    x = te[input_ids.reshape(-1)] + jnp.broadcast_to(
        params['pos_emb'][None], (B, S, D)).reshape(T, D)

    a1m = params['A1'].transpose(2, 3, 0, 1).reshape(R * 8, NB * 16)
    a2m = params['A2'].transpose(1, 3, 2, 0).reshape(R * 16 * 8, NB)
    b1m = params['B1'].transpose(2, 3, 0, 1).reshape(R * 32, NB * 8)
    b2m = params['B2'].transpose(1, 3, 2, 0).reshape(R * 8 * 32, NB)
    be = params['basis_emb']

    def v2(a):
        return a.reshape(1, -1)

    for p in params['layers']:
        n, q, k, v = _qkv(x, v2(p['n1s']), v2(p['n1b']), p['qw'], v2(p['qb']),
                          p['kw'], v2(p['kb']), p['vw'], v2(p['vb']))
        def heads(a):
            return a.reshape(B, S, H, DH).transpose(0, 2, 1, 3).reshape(B * H, S, DH)
        ctx = _attn(heads(q), heads(k), heads(v))
        ctx = ctx.reshape(B, H, S, DH).transpose(0, 2, 1, 3).reshape(T, D)
        wr = _router(n, ctx, p['sw'][:D], p['sw'][D:], v2(p['sb']),
                     p['recipes'], be)
        x = _ffn(x, wr, v2(p['n2s']), v2(p['n2b']), a1m, a2m, b1m, b2m,
                 p['wd'], v2(p['wdb']))

    logits = _head(x, v2(params['final_s']), v2(params['final_b']), te)
    return logits.reshape(B, S, V)


# speedup vs baseline: 3.9731x; 1.0114x over previous
import math

import jax
import jax.numpy as jnp
from jax.experimental import pallas as pl

V, D, DF = 8192, 256, 1024
NB, R, NN, KTOP, H, L = 32, 32, 64, 8, 4, 2
B, S = 2, 2048
DH = D // H
T = B * S

_NEG = -1e30


def _ln(x, s, b):
    m = jnp.mean(x, axis=-1, keepdims=True)
    xc = x - m
    v = jnp.mean(xc * xc, axis=-1, keepdims=True)
    return xc * jax.lax.rsqrt(v + 1e-5) * s + b


_TBQ = 512


def _qkv_body(x_ref, s_ref, b_ref, qw_ref, qb_ref, kw_ref, kb_ref,
              vw_ref, vb_ref, n_ref, q_ref, k_ref, v_ref):
    x = x_ref[...]
    n = _ln(x, s_ref[...], b_ref[...])
    n_ref[...] = n
    q_ref[...] = jnp.dot(n, qw_ref[...], preferred_element_type=jnp.float32) + qb_ref[...]
    k_ref[...] = jnp.dot(n, kw_ref[...], preferred_element_type=jnp.float32) + kb_ref[...]
    v_ref[...] = jnp.dot(n, vw_ref[...], preferred_element_type=jnp.float32) + vb_ref[...]


def _qkv(x, s, b, qw, qb, kw, kb, vw, vb):
    tok = pl.BlockSpec((_TBQ, D), lambda i: (i, 0))
    full = pl.BlockSpec((D, D), lambda i: (0, 0))
    vec = pl.BlockSpec((1, D), lambda i: (0, 0))
    return pl.pallas_call(
        _qkv_body,
        grid=(T // _TBQ,),
        in_specs=[tok, vec, vec, full, vec, full, vec, full, vec],
        out_specs=[tok, tok, tok, tok],
        out_shape=[jax.ShapeDtypeStruct((T, D), jnp.float32)] * 4,
    )(x, s, b, qw, qb, kw, kb, vw, vb)


_QB = 512


def _attn_body(q_ref, k_ref, v_ref, o_ref):
    qi = pl.program_id(1)
    q = q_ref[0]
    k = k_ref[0]
    v = v_ref[0]
    s = jax.lax.dot_general(q, k, (((1,), (1,)), ((), ())),
                            preferred_element_type=jnp.float32)
    s = s * (1.0 / math.sqrt(DH))
    row = qi * _QB + jax.lax.broadcasted_iota(jnp.int32, (_QB, S), 0)
    col = jax.lax.broadcasted_iota(jnp.int32, (_QB, S), 1)
    keep = col <= row
    s = jnp.where(keep, s, _NEG)
    mx = jnp.max(s, axis=-1, keepdims=True)
    e = jnp.exp(s - mx)
    e = jnp.where(keep, e, 0.0)
    p = e / jnp.sum(e, axis=-1, keepdims=True)
    o_ref[0] = jnp.dot(p, v, preferred_element_type=jnp.float32)


def _attn(q, k, v):
    qspec = pl.BlockSpec((1, _QB, DH), lambda bh, qi: (bh, qi, 0))
    kspec = pl.BlockSpec((1, S, DH), lambda bh, qi: (bh, 0, 0))
    return pl.pallas_call(
        _attn_body,
        grid=(B * H, S // _QB),
        in_specs=[qspec, kspec, kspec],
        out_specs=qspec,
        out_shape=jax.ShapeDtypeStruct((B * H, S, DH), jnp.float32),
    )(q, k, v)


_TBR = 512


def _router_body(n_ref, c_ref, sw1_ref, sw2_ref, sb_ref, rec_ref, be_ref,
                 wr_ref):
    n = n_ref[...]
    c = c_ref[...]
    query = (jnp.dot(n, sw1_ref[...], preferred_element_type=jnp.float32)
             + jnp.dot(c, sw2_ref[...], preferred_element_type=jnp.float32)
             + sb_ref[...])
    rec = rec_ref[...]
    rec_sm = jax.nn.softmax(rec, axis=-1)
    nemb = jnp.dot(rec_sm, be_ref[...], preferred_element_type=jnp.float32)
    scores = jax.lax.dot_general(query, nemb, (((1,), (1,)), ((), ())),
                                 preferred_element_type=jnp.float32)
    idx = jax.lax.broadcasted_iota(jnp.int32, scores.shape, 1)
    sel = jnp.zeros(scores.shape, jnp.bool_)
    work = scores
    for _ in range(KTOP):
        mx = jnp.max(work, axis=-1, keepdims=True)
        is_max = work == mx
        cand_idx = jnp.where(is_max, idx, NN)
        amin = jnp.min(cand_idx, axis=-1, keepdims=True)
        first = idx == amin
        sel = jnp.logical_or(sel, first)
        work = jnp.where(first, _NEG, work)
    smax = jnp.max(jnp.where(sel, scores, _NEG), axis=-1, keepdims=True)
    e = jnp.where(sel, jnp.exp(scores - smax), 0.0)
    w = e / jnp.sum(e, axis=-1, keepdims=True)
    wr_ref[...] = jnp.dot(w, rec_sm, preferred_element_type=jnp.float32)


def _router(n, c, sw1, sw2, sb, rec, be):
    tok = pl.BlockSpec((_TBR, D), lambda i: (i, 0))
    full = pl.BlockSpec((D, D), lambda i: (0, 0))
    vec = pl.BlockSpec((1, D), lambda i: (0, 0))
    recs = pl.BlockSpec((NN, NB), lambda i: (0, 0))
    bes = pl.BlockSpec((NB, D), lambda i: (0, 0))
    wrs = pl.BlockSpec((_TBR, NB), lambda i: (i, 0))
    return pl.pallas_call(
        _router_body,
        grid=(T // _TBR,),
        in_specs=[tok, tok, full, full, vec, recs, bes],
        out_specs=wrs,
        out_shape=jax.ShapeDtypeStruct((T, NB), jnp.float32),
    )(n, c, sw1, sw2, sb, rec, be)


_TBF = 128


def _tdot(a, b):
    return jax.lax.dot_general(a, b, (((0,), (0,)), ((), ())),
                               preferred_element_type=jnp.float32)


def _ffn_body(x_ref, wr_ref, s_ref, b_ref, a1_ref, a2_ref, b1_ref, b2_ref,
              wd_ref, wdb_ref, o_ref):
    x = x_ref[...]
    wr = wr_ref[...]
    wrt = wr.T
    n2 = _ln(x, s_ref[...], b_ref[...])

    xfm = n2.reshape(_TBF, 16, 16).transpose(1, 2, 0).reshape(16, 16 * _TBF)
    wtj = jnp.broadcast_to(wrt[:, None, :], (NB, 16, _TBF)).reshape(NB, 16 * _TBF)
    zt = (wtj[:, None, :] * xfm[None, :, :]).reshape(NB * 16, 16 * _TBF)
    t1t = jnp.dot(a1_ref[...].astype(jnp.bfloat16), zt.astype(jnp.bfloat16),
                  preferred_element_type=jnp.float32)
    ca2 = jnp.dot(a2_ref[...], wrt, preferred_element_type=jnp.float32)
    ca2 = ca2.reshape(R * 8, 16 * _TBF)
    t4 = t1t.reshape(R, 8, 1, 16 * _TBF)
    c4 = ca2.reshape(R, 1, 8, 16 * _TBF)
    prod = jnp.zeros((8, 8, 16 * _TBF), jnp.float32)
    for rc in range(4):
        rs = slice(rc * 8, (rc + 1) * 8)
        prod = prod + (t4[rs] * c4[rs]).sum(axis=0)
    h3 = prod.reshape(8, 8, 16, _TBF).sum(axis=2)

    hfm = h3.reshape(8, 8 * _TBF)
    wtj2 = jnp.broadcast_to(wrt[:, None, :], (NB, 8, _TBF)).reshape(NB, 8 * _TBF)
    zbt = (wtj2[:, None, :] * hfm[None, :, :]).reshape(NB * 8, 8 * _TBF)
    ut = jnp.dot(b1_ref[...].astype(jnp.bfloat16), zbt.astype(jnp.bfloat16),
                  preferred_element_type=jnp.float32)
    cb2 = jnp.dot(b2_ref[...], wrt, preferred_element_type=jnp.float32)
    cb2 = cb2.reshape(R * 32, 8 * _TBF)
    u4 = ut.reshape(R, 32, 1, 8 * _TBF)
    cb4 = cb2.reshape(R, 1, 32, 8 * _TBF)
    acc = jnp.zeros((32, 32, 8 * _TBF), jnp.float32)
    for rc in range(8):
        rs = slice(rc * 4, (rc + 1) * 4)
        acc = acc + (u4[rs] * cb4[rs]).sum(axis=0)
    out3 = acc.reshape(32, 32, 8, _TBF).sum(axis=2)
    ov = out3.reshape(DF, _TBF)

    g = 0.5 * ov * (1.0 + jax.lax.erf(ov * (1.0 / math.sqrt(2.0))))
    y = _tdot(g, wd_ref[...])
    o_ref[...] = x + y + wdb_ref[...]


def _ffn(x, wr, s, b, a1m, a2m, b1m, b2m, wd, wdb):
    tok = pl.BlockSpec((_TBF, D), lambda i: (i, 0))
    wrs = pl.BlockSpec((_TBF, NB), lambda i: (i, 0))
    vec = pl.BlockSpec((1, D), lambda i: (0, 0))
    a1s = pl.BlockSpec((R * 8, NB * 16), lambda i: (0, 0))
    a2s = pl.BlockSpec((R * 16 * 8, NB), lambda i: (0, 0))
    b1s = pl.BlockSpec((R * 32, NB * 8), lambda i: (0, 0))
    b2s = pl.BlockSpec((R * 8 * 32, NB), lambda i: (0, 0))
    wds = pl.BlockSpec((DF, D), lambda i: (0, 0))
    return pl.pallas_call(
        _ffn_body,
        grid=(T // _TBF,),
        in_specs=[tok, wrs, vec, vec, a1s, a2s, b1s, b2s, wds, vec],
        out_specs=tok,
        out_shape=jax.ShapeDtypeStruct((T, D), jnp.float32),
    )(x, wr, s, b, a1m, a2m, b1m, b2m, wd, wdb)


_TBH = 256


def _head_body(x_ref, s_ref, b_ref, te_ref, o_ref):
    xn = _ln(x_ref[...], s_ref[...], b_ref[...])
    o_ref[...] = jax.lax.dot_general(xn, te_ref[...], (((1,), (1,)), ((), ())),
                                     preferred_element_type=jnp.float32)


def _head(x, s, b, te):
    tok = pl.BlockSpec((_TBH, D), lambda i: (i, 0))
    vec = pl.BlockSpec((1, D), lambda i: (0, 0))
    tes = pl.BlockSpec((V, D), lambda i: (0, 0))
    outs = pl.BlockSpec((_TBH, V), lambda i: (i, 0))
    return pl.pallas_call(
        _head_body,
        grid=(T // _TBH,),
        in_specs=[tok, vec, vec, tes],
        out_specs=outs,
        out_shape=jax.ShapeDtypeStruct((T, V), jnp.float32),
    )(x, s, b, te)


def kernel(input_ids, params):
    te = params['token_emb']
    x = te[input_ids.reshape(-1)] + jnp.broadcast_to(
        params['pos_emb'][None], (B, S, D)).reshape(T, D)

    a1m = params['A1'].transpose(2, 3, 0, 1).reshape(R * 8, NB * 16)
    a2m = params['A2'].transpose(1, 3, 2, 0).reshape(R * 16 * 8, NB)
    b1m = params['B1'].transpose(2, 3, 0, 1).reshape(R * 32, NB * 8)
    b2m = params['B2'].transpose(1, 3, 2, 0).reshape(R * 8 * 32, NB)
    be = params['basis_emb']

    def v2(a):
        return a.reshape(1, -1)

    for p in params['layers']:
        n, q, k, v = _qkv(x, v2(p['n1s']), v2(p['n1b']), p['qw'], v2(p['qb']),
                          p['kw'], v2(p['kb']), p['vw'], v2(p['vb']))
        def heads(a):
            return a.reshape(B, S, H, DH).transpose(0, 2, 1, 3).reshape(B * H, S, DH)
        ctx = _attn(heads(q), heads(k), heads(v))
        ctx = ctx.reshape(B, H, S, DH).transpose(0, 2, 1, 3).reshape(T, D)
        wr = _router(n, ctx, p['sw'][:D], p['sw'][D:], v2(p['sb']),
                     p['recipes'], be)
        x = _ffn(x, wr, v2(p['n2s']), v2(p['n2b']), a1m, a2m, b1m, b2m,
                 p['wd'], v2(p['wdb']))

    logits = _head(x, v2(params['final_s']), v2(params['final_b']), te)
    return logits.reshape(B, S, V)
